# baseline (speedup 1.0000x reference)
import jax
import jax.numpy as jnp
from jax import lax
from jax.experimental import pallas as pl
from jax.experimental.pallas import tpu as pltpu
from jax.experimental.pallas import tpu_sc as plsc

RES = 2048
CH = 8
B = 4
HW = 512 * 512
NPTS = B * HW
NW = 32
PER_W = NPTS // NW
P = 1024
NCHUNK = PER_W // P
NGRP = P // 16
NBLK = P // 128

_OUT_TYPE = jax.ShapeDtypeStruct((B, CH, 512, 512), jnp.float32)
NROW = P // 512

_SCRATCH = [
    pltpu.VMEM((2 * P,), jnp.float32),
    pltpu.VMEM((4 * NBLK, 128), jnp.int32),
    pltpu.VMEM((P, CH), jnp.float32),
    pltpu.VMEM((P, CH), jnp.float32),
    pltpu.VMEM((P, CH), jnp.float32),
    pltpu.VMEM((P, CH), jnp.float32),
    pltpu.VMEM((2, P), jnp.float32),
    pltpu.VMEM((CH, NROW, 512), jnp.float32),
    pltpu.SemaphoreType.DMA,
]


def _sc_body(uv_hbm, tex_hbm, out_hbm, uv_v, idx_v, g00, g01, g10, g11,
             w_v, ob_v, sem):
    wid = lax.axis_index("s") * 2 + lax.axis_index("c")
    lane = lax.iota(jnp.int32, 16)

    def chunk_body(chunk, carry):
        gbase = wid * PER_W + chunk * P

        pltpu.sync_copy(uv_hbm.at[pl.ds(gbase * 2, 2 * P)], uv_v)

        def index_body(j, carry):
            o2 = j * 32
            u = plsc.load_gather(uv_v, [lane * 2 + o2])
            v = plsc.load_gather(uv_v, [lane * 2 + o2 + 1])
            fu = u - u.astype(jnp.int32).astype(jnp.float32)
            fv = v - v.astype(jnp.int32).astype(jnp.float32)
            x = ((fu * 2.0 - 1.0) + 1.0) * 0.5 * float(RES - 1)
            y = ((fv * 2.0 - 1.0) + 1.0) * 0.5 * float(RES - 1)
            xi = x.astype(jnp.int32)
            yi = y.astype(jnp.int32)
            wx = x - xi.astype(jnp.float32)
            wy = y - yi.astype(jnp.float32)
            i00 = yi * RES + xi
            m = j // 8
            o = (j % 8) * 16
            idx_v[m, pl.ds(o, 16)] = i00
            idx_v[NBLK + m, pl.ds(o, 16)] = i00 + 1
            idx_v[2 * NBLK + m, pl.ds(o, 16)] = i00 + RES
            idx_v[3 * NBLK + m, pl.ds(o, 16)] = i00 + RES + 1
            o16 = j * 16
            w_v[0, pl.ds(o16, 16)] = wx
            w_v[1, pl.ds(o16, 16)] = wy
            return carry

        lax.fori_loop(0, NGRP, index_body, 0)

        copies = []
        for k, g in enumerate((g00, g01, g10, g11)):
            for m in range(NBLK):
                copies.append(pltpu.async_copy(
                    tex_hbm.at[idx_v.at[k * NBLK + m]],
                    g.at[pl.ds(m * 128, 128)], sem))
        for cp in copies:
            cp.wait()

        def blend_body(j, carry):
            o16 = j * 16
            wx = w_v[0, pl.ds(o16, 16)]
            wy = w_v[1, pl.ds(o16, 16)]
            mx = 1.0 - wx
            my = 1.0 - wy
            w00 = mx * my
            w01 = wx * my
            w10 = mx * wy
            w11 = wx * wy
            pidx = lane + o16
            for c in range(CH):
                cidx = jnp.full((16,), c, jnp.int32)
                v00 = plsc.load_gather(g00, [pidx, cidx])
                v01 = plsc.load_gather(g01, [pidx, cidx])
                v10 = plsc.load_gather(g10, [pidx, cidx])
                v11 = plsc.load_gather(g11, [pidx, cidx])
                ob_v[c, j // 32, pl.ds((j % 32) * 16, 16)] = (
                    w00 * v00 + w01 * v01 + w10 * v10 + w11 * v11)
            return carry

        lax.fori_loop(0, NGRP, blend_body, 0)

        b = gbase // HW
        r0 = (gbase % HW) // 512
        for c in range(CH):
            pltpu.sync_copy(ob_v.at[c], out_hbm.at[b, c, pl.ds(r0, NROW)])
        return carry

    lax.fori_loop(0, NCHUNK, chunk_body, 0)


_CACHE = {}


def _sc_grid_sample():
    if "fn" not in _CACHE:
        mesh = plsc.VectorSubcoreMesh(core_axis_name="c", subcore_axis_name="s")
        _CACHE["fn"] = pl.kernel(
            _sc_body, out_type=_OUT_TYPE, mesh=mesh, scratch_types=_SCRATCH,
            compiler_params=pltpu.CompilerParams(
                needs_layout_passes=False, use_tc_tiling_on_sc=False))
    return _CACHE["fn"]


def kernel(uvs, tex):
    uv_flat = uvs.reshape(-1)
    tex_t = tex.reshape(CH, RES * RES).T
    return _sc_grid_sample()(uv_flat, tex_t)


# speedup vs baseline: 2.7436x; 2.7436x over previous
import jax
import jax.numpy as jnp
from jax import lax
from jax.experimental import pallas as pl
from jax.experimental.pallas import tpu as pltpu
from jax.experimental.pallas import tpu_sc as plsc

RES = 2048
CH = 8
B = 4
HW = 512 * 512
NPTS = B * HW
NW = 32

_PARAMS = pltpu.CompilerParams(needs_layout_passes=False,
                               use_tc_tiling_on_sc=False)


def _wid():
    return lax.axis_index("s") * 2 + lax.axis_index("c")


NUNIT = 256 * 16
UPW = NUNIT // NW


def _tr_body(tex4_hbm, out_hbm, in_v, out_v, sem):
    w = _wid()
    lane = lax.iota(jnp.int32, 16)
    cvec = lane % 8
    dvec = lane // 8
    base0 = cvec * 1024 + dvec

    def unit_body(u, carry):
        uid = w * UPW + u
        yt = uid // 16
        xt = uid % 16
        cps = [pltpu.async_copy(tex4_hbm.at[c, yt, xt],
                                in_v.at[pl.ds(c * 1024, 1024)], sem)
               for c in range(CH)]
        for cp in cps:
            cp.wait()
        for ys in range(8):
            bvec = base0 + ys * 128

            def shuf(g, idxv):
                val = plsc.load_gather(in_v, [idxv])
                out_v[ys, g // 8, pl.ds((g % 8) * 16, 16)] = val
                return idxv + 2

            lax.fori_loop(0, 64, shuf, bvec)
            r0 = (yt * 8 + ys) * 128 + xt * 8
            pltpu.sync_copy(out_v.at[ys], out_hbm.at[pl.ds(r0, 8)])
        return carry

    lax.fori_loop(0, UPW, unit_body, 0)


P = 1024
NGRP = P // 16
NBLK = P // 128
SUPER = 4 * P
NSUP = NPTS // NW // SUPER


def _gs_body(uv_hbm, tex_hbm, out_hbm, u_v, v_v, idx_v, g00, g01, g10, g11,
             w_v, ob_v, sem):
    w = _wid()
    lane = lax.iota(jnp.int32, 16)

    def super_body(s, carry):
        rtg = w * NSUP + s
        b = rtg // 64
        rt = rtg % 64
        for sub in range(4):
            i = rt * 8 + sub * 2
            cps = []
            for r in range(2):
                for k in range(4):
                    dst = pl.ds(r * 512 + k * 128, 128)
                    cps.append(pltpu.async_copy(
                        uv_hbm.at[b, i + r, k, 0], u_v.at[dst], sem))
                    cps.append(pltpu.async_copy(
                        uv_hbm.at[b, i + r, k, 1], v_v.at[dst], sem))
            for cp in cps:
                cp.wait()

            def index_body(j, carry):
                o16 = j * 16
                u = u_v[pl.ds(o16, 16)]
                v = v_v[pl.ds(o16, 16)]
                fu = u - u.astype(jnp.int32).astype(jnp.float32)
                fv = v - v.astype(jnp.int32).astype(jnp.float32)
                x = ((fu * 2.0 - 1.0) + 1.0) * 0.5 * float(RES - 1)
                y = ((fv * 2.0 - 1.0) + 1.0) * 0.5 * float(RES - 1)
                xi = x.astype(jnp.int32)
                yi = y.astype(jnp.int32)
                wx = x - xi.astype(jnp.float32)
                wy = y - yi.astype(jnp.float32)
                i00 = yi * RES + xi
                m = j // 8
                o = (j % 8) * 16
                idx_v[m, pl.ds(o, 16)] = i00
                idx_v[NBLK + m, pl.ds(o, 16)] = i00 + 1
                idx_v[2 * NBLK + m, pl.ds(o, 16)] = i00 + RES
                idx_v[3 * NBLK + m, pl.ds(o, 16)] = i00 + RES + 1
                w_v[0, pl.ds(o16, 16)] = wx
                w_v[1, pl.ds(o16, 16)] = wy
                return carry

            lax.fori_loop(0, NGRP, index_body, 0)

            cps = []
            for k, g in enumerate((g00, g01, g10, g11)):
                for m in range(NBLK):
                    cps.append(pltpu.async_copy(
                        tex_hbm.at[idx_v.at[k * NBLK + m]],
                        g.at[pl.ds(m * 128, 128)], sem))
            for cp in cps:
                cp.wait()

            def blend_body(j, carry):
                o16 = j * 16
                wx = w_v[0, pl.ds(o16, 16)]
                wy = w_v[1, pl.ds(o16, 16)]
                mx = 1.0 - wx
                my = 1.0 - wy
                w00 = mx * my
                w01 = wx * my
                w10 = mx * wy
                w11 = wx * wy
                pidx = lane + o16
                ys = sub * 2 + j // 32
                col = (j % 32) * 16
                for c in range(CH):
                    ci = jnp.full((16,), c, jnp.int32)
                    v00 = plsc.load_gather(g00, [pidx, ci])
                    v01 = plsc.load_gather(g01, [pidx, ci])
                    v10 = plsc.load_gather(g10, [pidx, ci])
                    v11 = plsc.load_gather(g11, [pidx, ci])
                    ob_v[c, ys, pl.ds(col, 16)] = (w00 * v00 + w01 * v01
                                                   + w10 * v10 + w11 * v11)
                return carry

            lax.fori_loop(0, NGRP, blend_body, 0)

        for c in range(CH):
            for ct in range(4):
                pltpu.sync_copy(ob_v.at[c, :, pl.ds(ct * 128, 128)],
                                out_hbm.at[b, c, rt, ct])
        return carry

    lax.fori_loop(0, NSUP, super_body, 0)


_CACHE = {}


def _build():
    if "fns" not in _CACHE:
        mesh = plsc.VectorSubcoreMesh(core_axis_name="c", subcore_axis_name="s")
        tr = pl.kernel(
            _tr_body,
            out_type=jax.ShapeDtypeStruct((RES * RES // 16, 128), jnp.float32),
            mesh=mesh,
            scratch_types=[
                pltpu.VMEM((CH * 8 * 128,), jnp.float32),
                pltpu.VMEM((8, 8, 128), jnp.float32),
                pltpu.SemaphoreType.DMA,
            ],
            compiler_params=_PARAMS)
        gs = pl.kernel(
            _gs_body,
            out_type=jax.ShapeDtypeStruct((B, CH, 64, 4, 8, 128), jnp.float32),
            mesh=mesh,
            scratch_types=[
                pltpu.VMEM((P,), jnp.float32),
                pltpu.VMEM((P,), jnp.float32),
                pltpu.VMEM((4 * NBLK, 128), jnp.int32),
                pltpu.VMEM((P, CH), jnp.float32),
                pltpu.VMEM((P, CH), jnp.float32),
                pltpu.VMEM((P, CH), jnp.float32),
                pltpu.VMEM((P, CH), jnp.float32),
                pltpu.VMEM((2, P), jnp.float32),
                pltpu.VMEM((CH, 8, 512), jnp.float32),
                pltpu.SemaphoreType.DMA,
            ],
            compiler_params=_PARAMS)
        _CACHE["fns"] = (tr, gs)
    return _CACHE["fns"]


def kernel(uvs, tex):
    tr, gs = _build()
    tex4 = (tex.reshape(CH, 256, 8, 16, 128).transpose(0, 1, 3, 2, 4)
            .reshape(CH, 256, 16, 1024))
    uv_phys = uvs.reshape(B, 512, 4, 128, 2).transpose(0, 1, 2, 4, 3)
    tex_t = tr(tex4).reshape(RES * RES, CH)
    out6 = gs(uv_phys, tex_t)
    return out6.transpose(0, 1, 2, 4, 3, 5).reshape(B, CH, 512, 512)


# speedup vs baseline: 3.9320x; 1.4332x over previous
import jax
import jax.numpy as jnp
from jax import lax
from jax.experimental import pallas as pl
from jax.experimental.pallas import tpu as pltpu
from jax.experimental.pallas import tpu_sc as plsc

RES = 2048
CH = 8
B = 4
HW = 512 * 512
NPTS = B * HW
NW = 32

_PARAMS = pltpu.CompilerParams(needs_layout_passes=False,
                               use_tc_tiling_on_sc=False)


def _wid():
    return lax.axis_index("s") * 2 + lax.axis_index("c")


NUNIT = 256 * 16
UPW = NUNIT // NW


def _tr_body(tex4_hbm, out_hbm, in_v, out_v, semi0, semi1, semo0, semo1):
    w = _wid()
    lane = lax.iota(jnp.int32, 16)
    cvec = lane % 8
    dvec = lane // 8

    def fire_in(u, p, sem):
        uid = jnp.minimum(w * UPW + u, NUNIT - 1)
        yt = uid // 16
        xt = uid % 16
        return pltpu.async_copy(tex4_hbm.at[:, yt, xt], in_v.at[p], sem)

    def shuffle(p):
        for ys in range(8):
            def shuf(g, carry, ys=ys, p=p):
                xv = dvec + (ys * 128 + g * 2)
                val = plsc.load_gather(in_v.at[p], [cvec, xv])
                out_v[p, ys, pl.ds(g * 16, 16)] = val
                return carry

            lax.fori_loop(0, 64, shuf, 0, unroll=8)

    def fire_out(u, p, sem):
        uid = w * UPW + u
        yt = uid // 16
        xt = uid % 16
        return pltpu.async_copy(out_v.at[p],
                                out_hbm.at[pl.ds(yt * 8, 8), xt], sem)

    fire_in(0, 0, semi0)
    fire_in(1, 1, semi1)

    def pair_body(i, carry):
        u0 = 2 * i
        pltpu.make_async_copy(tex4_hbm.at[:, 0, 0], in_v.at[0], semi0).wait()

        @pl.when(i > 0)
        def _():
            pltpu.make_async_copy(out_v.at[0],
                                  out_hbm.at[pl.ds(0, 8), 0], semo0).wait()

        shuffle(0)
        fire_in(u0 + 2, 0, semi0)
        fire_out(u0, 0, semo0)
        pltpu.make_async_copy(tex4_hbm.at[:, 0, 0], in_v.at[1], semi1).wait()

        @pl.when(i > 0)
        def _():
            pltpu.make_async_copy(out_v.at[1],
                                  out_hbm.at[pl.ds(0, 8), 0], semo1).wait()

        shuffle(1)
        fire_in(u0 + 3, 1, semi1)
        fire_out(u0 + 1, 1, semo1)
        return carry

    lax.fori_loop(0, UPW // 2, pair_body, 0)
    pltpu.make_async_copy(tex4_hbm.at[:, 0, 0], in_v.at[0], semi0).wait()
    pltpu.make_async_copy(tex4_hbm.at[:, 0, 0], in_v.at[1], semi1).wait()
    pltpu.make_async_copy(out_v.at[0], out_hbm.at[pl.ds(0, 8), 0], semo0).wait()
    pltpu.make_async_copy(out_v.at[1], out_hbm.at[pl.ds(0, 8), 0], semo1).wait()


P = 512
SUBS = 8
NSUP = NPTS // NW // (SUBS * P)
NGRP = SUBS * P // 16


def _gs_body(uv_hbm, tex_hbm, out_hbm, u_v, v_v, idx_v, gA, gB,
             w_v, ob_v, semu, semA, semB, semo):
    w = _wid()
    lane = lax.iota(jnp.int32, 16)

    def fire_gathers(sub, buf, sem):
        cps = []
        for k in range(4):
            for m in range(4):
                cps.append(pltpu.async_copy(
                    tex_hbm.at[idx_v.at[sub * 16 + k * 4 + m]],
                    buf.at[k, pl.ds(m * 128, 128)], sem))
        return cps

    def drain_gathers(buf, sem):
        for k in range(4):
            pltpu.make_async_copy(
                tex_hbm.at[pl.ds(0, P)], buf.at[k], sem).wait()

    def super_body(s, carry):
        rtg = w * NSUP + s
        b = rtg // 64
        rt = rtg % 64
        i0 = rt * 8

        cps = []
        for k in range(4):
            cps.append(pltpu.async_copy(
                uv_hbm.at[b, pl.ds(i0, 8), k, 0], u_v.at[k], semu))
            cps.append(pltpu.async_copy(
                uv_hbm.at[b, pl.ds(i0, 8), k, 1], v_v.at[k], semu))
        for cp in cps:
            cp.wait()

        @pl.when(s > 0)
        def _():
            for _i in range(32):
                pltpu.make_async_copy(
                    uv_hbm.at[0, pl.ds(0, 8), 0, 0],
                    ob_v.at[0, :, pl.ds(0, 128)], semo).wait()

        def index_body(j, carry):
            sub = j // 32
            jj = j % 32
            k4 = (j // 8) % 4
            col = (j % 8) * 16
            u = u_v[k4, sub, pl.ds(col, 16)]
            v = v_v[k4, sub, pl.ds(col, 16)]
            fu = u - u.astype(jnp.int32).astype(jnp.float32)
            fv = v - v.astype(jnp.int32).astype(jnp.float32)
            x = ((fu * 2.0 - 1.0) + 1.0) * 0.5 * float(RES - 1)
            y = ((fv * 2.0 - 1.0) + 1.0) * 0.5 * float(RES - 1)
            xi = x.astype(jnp.int32)
            yi = y.astype(jnp.int32)
            wx = x - xi.astype(jnp.float32)
            wy = y - yi.astype(jnp.float32)
            i00 = yi * RES + xi
            m = jj // 8
            o = (jj % 8) * 16
            r0 = sub * 16 + m
            idx_v[r0, pl.ds(o, 16)] = i00
            idx_v[r0 + 4, pl.ds(o, 16)] = i00 + 1
            idx_v[r0 + 8, pl.ds(o, 16)] = i00 + RES
            idx_v[r0 + 12, pl.ds(o, 16)] = i00 + RES + 1
            o16 = j * 16
            w_v[0, pl.ds(o16, 16)] = wx
            w_v[1, pl.ds(o16, 16)] = wy
            return carry

        lax.fori_loop(0, NGRP, index_body, 0, unroll=2)

        fire_gathers(0, gA, semA)
        fire_gathers(1, gB, semB)
        for sub in range(SUBS):
            buf, sem = (gA, semA) if sub % 2 == 0 else (gB, semB)
            drain_gathers(buf, sem)

            def blend_body(j2, carry, sub=sub, buf=buf):
                o16 = j2 * 16
                wbase = sub * P + o16
                wx = w_v[0, pl.ds(wbase, 16)]
                wy = w_v[1, pl.ds(wbase, 16)]
                mx = 1.0 - wx
                my = 1.0 - wy
                w00 = mx * my
                w01 = wx * my
                w10 = mx * wy
                w11 = wx * wy
                pidx = lane + o16
                for c in range(CH):
                    ci = jnp.full((16,), c, jnp.int32)
                    v00 = plsc.load_gather(buf.at[0], [pidx, ci])
                    v01 = plsc.load_gather(buf.at[1], [pidx, ci])
                    v10 = plsc.load_gather(buf.at[2], [pidx, ci])
                    v11 = plsc.load_gather(buf.at[3], [pidx, ci])
                    ob_v[c, sub, pl.ds(o16, 16)] = (w00 * v00 + w01 * v01
                                                    + w10 * v10 + w11 * v11)
                return carry

            lax.fori_loop(0, P // 16, blend_body, 0, unroll=2)
            if sub + 2 < SUBS:
                fire_gathers(sub + 2, buf, sem)

        for c in range(CH):
            for ct in range(4):
                pltpu.async_copy(ob_v.at[c, :, pl.ds(ct * 128, 128)],
                                 out_hbm.at[b, c, rt, ct], semo)
        return carry

    lax.fori_loop(0, NSUP, super_body, 0)
    for _i in range(32):
        pltpu.make_async_copy(uv_hbm.at[0, pl.ds(0, 8), 0, 0],
                              ob_v.at[0, :, pl.ds(0, 128)], semo).wait()


_CACHE = {}


def _build():
    if "fns" not in _CACHE:
        mesh = plsc.VectorSubcoreMesh(core_axis_name="c", subcore_axis_name="s")
        tr = pl.kernel(
            _tr_body,
            out_type=jax.ShapeDtypeStruct((RES, 16, 1024), jnp.float32),
            mesh=mesh,
            scratch_types=[
                pltpu.VMEM((2, CH, 1024), jnp.float32),
                pltpu.VMEM((2, 8, 1024), jnp.float32),
                pltpu.SemaphoreType.DMA,
                pltpu.SemaphoreType.DMA,
                pltpu.SemaphoreType.DMA,
                pltpu.SemaphoreType.DMA,
            ],
            compiler_params=_PARAMS)
        gs = pl.kernel(
            _gs_body,
            out_type=jax.ShapeDtypeStruct((B, CH, 64, 4, 8, 128), jnp.float32),
            mesh=mesh,
            scratch_types=[
                pltpu.VMEM((4, 8, 128), jnp.float32),
                pltpu.VMEM((4, 8, 128), jnp.float32),
                pltpu.VMEM((128, 128), jnp.int32),
                pltpu.VMEM((4, P, CH), jnp.float32),
                pltpu.VMEM((4, P, CH), jnp.float32),
                pltpu.VMEM((2, SUBS * P), jnp.float32),
                pltpu.VMEM((CH, SUBS, P), jnp.float32),
                pltpu.SemaphoreType.DMA,
                pltpu.SemaphoreType.DMA,
                pltpu.SemaphoreType.DMA,
                pltpu.SemaphoreType.DMA,
            ],
            compiler_params=_PARAMS)
        _CACHE["fns"] = (tr, gs)
    return _CACHE["fns"]


def kernel(uvs, tex):
    tr, gs = _build()
    tex4 = (tex.reshape(CH, 256, 8, 16, 128).transpose(0, 1, 3, 2, 4)
            .reshape(CH, 256, 16, 1024))
    uv_phys = uvs.reshape(B, 512, 4, 128, 2).transpose(0, 1, 2, 4, 3)
    tex_t = tr(tex4).reshape(RES * RES, CH)
    out6 = gs(uv_phys, tex_t)
    return out6.transpose(0, 1, 2, 4, 3, 5).reshape(B, CH, 512, 512)
